# baseline (speedup 1.0000x reference)
import functools

import jax
import jax.numpy as jnp
from jax import lax
from jax.experimental import pallas as pl
from jax.experimental.pallas import tpu as pltpu
from jax.experimental.pallas import tpu_sc as plsc

D = 64
CB = 16


def _make_encoder(batch, seq):
    info = plsc.get_sparse_core_info()
    nc, ns, lanes = info.num_cores, info.num_subcores, info.num_lanes
    nw = nc * ns
    b_per_w = batch // nw
    assert batch % nw == 0 and b_per_w % CB == 0 and CB == lanes
    n_chunks = b_per_w // CB
    row_w = seq * D

    mesh = plsc.VectorSubcoreMesh(core_axis_name="c", subcore_axis_name="s")

    @functools.partial(
        pl.kernel,
        mesh=mesh,
        compiler_params=pltpu.CompilerParams(
            use_tc_tiling_on_sc=False, needs_layout_passes=False),
        out_type=jax.ShapeDtypeStruct((seq, D, batch), jnp.float32),
        scratch_types=[
            pltpu.VMEM((CB, seq), jnp.int32),
            pltpu.VMEM((CB, seq), jnp.int32),
            pltpu.VMEM((CB, seq, D), jnp.float32),
            pltpu.VMEM((seq, D, CB), jnp.float32),
            pltpu.SemaphoreType.DMA,
            pltpu.SemaphoreType.DMA,
            pltpu.SemaphoreType.DMA,
        ],
    )
    def enc(node_hbm, deg_hbm, nidx_hbm, didx_hbm, out_hbm,
            nidx_v, didx_v, rows_v, tr_v, nsem, dsem, osem):
        wid = lax.axis_index("s") * nc + lax.axis_index("c")
        base = wid * b_per_w

        def chunk_body(ci, carry):
            b0 = base + ci * CB
            pltpu.sync_copy(nidx_hbm.at[pl.ds(b0, CB)], nidx_v)
            pltpu.sync_copy(didx_hbm.at[pl.ds(b0, CB)], didx_v)
            copies = []
            for j in range(CB):
                copies.append(pltpu.async_copy(
                    node_hbm.at[nidx_v.at[j]], rows_v.at[j], nsem))
            for cp in copies:
                cp.wait()
            copies = []
            for j in range(CB):
                copies.append(pltpu.async_copy(
                    deg_hbm.at[didx_v.at[j]], rows_v.at[j], dsem, add=True))
            for cp in copies:
                cp.wait()

            def s_body(s, c2):
                lane_base = lax.iota(jnp.int32, CB) * row_w
                zeros = jnp.zeros((CB,), jnp.int32)
                idx = lane_base + s * D
                for d in range(D):
                    vals = plsc.load_gather(rows_v, [zeros, zeros, idx + d])
                    tr_v[s, d, :] = vals
                return c2

            lax.fori_loop(0, seq, s_body, 0)
            pltpu.async_copy(
                tr_v, out_hbm.at[:, :, pl.ds(b0, CB)], osem).wait()
            return carry

        lax.fori_loop(0, n_chunks, chunk_body, 0)

    return enc


@jax.jit
def kernel(user_seq, user_degree, node_table, degree_table):
    b, s = user_seq.shape
    enc = _make_encoder(b, s)
    out_t = enc(node_table, degree_table, user_seq, user_degree)
    return jnp.transpose(out_t, (2, 0, 1))


# speedup vs baseline: 1.2983x; 1.2983x over previous
import functools

import jax
import jax.numpy as jnp
from jax import lax
from jax.experimental import pallas as pl
from jax.experimental.pallas import tpu as pltpu
from jax.experimental.pallas import tpu_sc as plsc

D = 64
CB = 16


def _make_encoder(batch, seq):
    info = plsc.get_sparse_core_info()
    nc, ns, lanes = info.num_cores, info.num_subcores, info.num_lanes
    nw = nc * ns
    b_per_w = batch // nw
    assert batch % nw == 0 and b_per_w % CB == 0 and CB == lanes
    n_chunks = b_per_w // CB
    row_w = seq * D

    mesh = plsc.VectorSubcoreMesh(core_axis_name="c", subcore_axis_name="s")

    @functools.partial(
        pl.kernel,
        mesh=mesh,
        compiler_params=pltpu.CompilerParams(
            use_tc_tiling_on_sc=False, needs_layout_passes=False),
        out_type=jax.ShapeDtypeStruct((seq, D, batch), jnp.float32),
        scratch_types=[
            pltpu.VMEM((CB, seq), jnp.int32),
            pltpu.VMEM((CB, seq), jnp.int32),
            pltpu.VMEM((CB, seq, D), jnp.float32),
            pltpu.VMEM((seq, D, CB), jnp.float32),
            pltpu.SemaphoreType.DMA,
            pltpu.SemaphoreType.DMA,
            pltpu.SemaphoreType.DMA,
        ],
    )
    def enc(node_hbm, deg_hbm, nidx_hbm, didx_hbm, out_hbm,
            nidx_v, didx_v, rows_v, tr_v, nsem, dsem, osem):
        wid = lax.axis_index("s") * nc + lax.axis_index("c")
        base = wid * b_per_w

        def chunk_body(ci, carry):
            b0 = base + ci * CB
            pltpu.sync_copy(nidx_hbm.at[pl.ds(b0, CB)], nidx_v)
            pltpu.sync_copy(didx_hbm.at[pl.ds(b0, CB)], didx_v)
            copies = []
            for j in range(CB):
                copies.append(pltpu.async_copy(
                    node_hbm.at[nidx_v.at[j]], rows_v.at[j], nsem))
            for cp in copies:
                cp.wait()
            copies = []
            for j in range(CB):
                copies.append(pltpu.async_copy(
                    deg_hbm.at[didx_v.at[j]], rows_v.at[j], dsem, add=True))
            for cp in copies:
                cp.wait()

            def s_body(s, c2):
                lane_base = lax.iota(jnp.int32, CB) * row_w
                zeros = jnp.zeros((CB,), jnp.int32)
                idx = lane_base + s * D
                unroll = 8
                for d0 in range(0, D, unroll):
                    vals = [
                        plsc.load_gather(rows_v, [zeros, zeros, idx + d0 + u])
                        for u in range(unroll)
                    ]
                    for u in range(unroll):
                        tr_v[s, d0 + u, :] = vals[u]
                return c2

            lax.fori_loop(0, seq, s_body, 0)
            pltpu.async_copy(
                tr_v, out_hbm.at[:, :, pl.ds(b0, CB)], osem).wait()
            return carry

        lax.fori_loop(0, n_chunks, chunk_body, 0)

    return enc


@jax.jit
def kernel(user_seq, user_degree, node_table, degree_table):
    b, s = user_seq.shape
    enc = _make_encoder(b, s)
    out_t = enc(node_table, degree_table, user_seq, user_degree)
    return jnp.transpose(out_t, (2, 0, 1))


# speedup vs baseline: 1.5903x; 1.2249x over previous
import functools

import jax
import jax.numpy as jnp
from jax import lax
from jax.experimental import pallas as pl
from jax.experimental.pallas import tpu as pltpu
from jax.experimental.pallas import tpu_sc as plsc

D = 64
CB = 16


def _make_encoder(batch, seq):
    info = plsc.get_sparse_core_info()
    nc, ns = info.num_cores, info.num_subcores
    nw = nc * ns
    b_per_w = batch // nw
    assert batch % nw == 0 and b_per_w % CB == 0
    n_chunks = b_per_w // CB
    chunk_128rows = CB * seq * D // 128

    mesh = plsc.VectorSubcoreMesh(core_axis_name="c", subcore_axis_name="s")

    @functools.partial(
        pl.kernel,
        mesh=mesh,
        compiler_params=pltpu.CompilerParams(use_tc_tiling_on_sc=False),
        out_type=jax.ShapeDtypeStruct((batch, seq, D), jnp.float32),
        scratch_types=[
            pltpu.VMEM((CB, seq), jnp.int32),
            pltpu.VMEM((CB, seq), jnp.int32),
            pltpu.VMEM((CB, seq, D), jnp.float32),
            pltpu.SemaphoreType.DMA,
            pltpu.SemaphoreType.DMA,
        ],
    )
    def enc(node_hbm, deg_hbm, nidx_hbm, didx_hbm, out_hbm,
            nidx_v, didx_v, rows_v, nsem, dsem):
        wid = lax.axis_index("s") * nc + lax.axis_index("c")
        base = wid * b_per_w

        def chunk_body(ci, carry):
            b0 = base + ci * CB
            pltpu.sync_copy(nidx_hbm.at[pl.ds(b0, CB)], nidx_v)
            pltpu.sync_copy(didx_hbm.at[pl.ds(b0, CB)], didx_v)
            copies = []
            for j in range(CB):
                copies.append(pltpu.async_copy(
                    node_hbm.at[nidx_v.at[j]], rows_v.at[j], nsem))
            for cp in copies:
                cp.wait()
            copies = []
            for j in range(CB):
                copies.append(pltpu.async_copy(
                    deg_hbm.at[didx_v.at[j]], rows_v.at[j], dsem, add=True))
            for cp in copies:
                cp.wait()
            pltpu.sync_copy(rows_v, out_hbm.at[pl.ds(b0, CB)])
            return carry

        lax.fori_loop(0, n_chunks, chunk_body, 0)

    return enc


@jax.jit
def kernel(user_seq, user_degree, node_table, degree_table):
    b, s = user_seq.shape
    n_split = 4
    bs = b // n_split
    enc = _make_encoder(bs, s)
    outs = [
        enc(node_table, degree_table,
            user_seq[i * bs:(i + 1) * bs], user_degree[i * bs:(i + 1) * bs])
        for i in range(n_split)
    ]
    return jnp.concatenate(outs, axis=0)
